# baseline (speedup 1.0000x reference)
import jax
import jax.numpy as jnp
from jax import lax
from jax.experimental import pallas as pl
from jax.experimental.pallas import tpu as pltpu
from jax.experimental.pallas import tpu_sc as plsc

D_MODEL = 64
SCALE = 8.0
NC, NS = 2, 16
NW = NC * NS
ROWS = 4096 * 200
BPW = ROWS // NW
G = 2
CHUNK = G * 128
NCH = BPW // CHUNK
PAIRS = NCH // 2
MAJ_PER_W = BPW // 128


def _emb_body(table, idx, out, idx_v, rows_v, wbuf, gs0, gs1, ws0, ws1):
    wid = lax.axis_index("s") * NC + lax.axis_index("c")
    mbase = wid * MAJ_PER_W

    pltpu.sync_copy(idx.at[pl.ds(mbase, MAJ_PER_W)], idx_v)

    def fire_gathers(c, b, gsem):
        for t in range(G):
            pltpu.async_copy(
                table.at[idx_v.at[c * G + t]], rows_v.at[b, t], gsem
            )

    fire_gathers(0, 0, gs0)
    fire_gathers(1, 1, gs1)

    def pair_body(k, carry):
        for b, gsem, wsem in ((0, gs0, ws0), (1, gs1, ws1)):
            c = 2 * k + b

            for t in range(G):
                pltpu.make_async_copy(
                    table.at[pl.ds(0, 128)], rows_v.at[b, t], gsem
                ).wait()

            @pl.when(k >= 1)
            def _drain_write():
                pltpu.make_async_copy(
                    wbuf.at[b], out.at[pl.ds(mbase, G)], wsem
                ).wait()

            @plsc.parallel_loop(0, 128, unroll=4)
            def _scale(i):
                for t in range(G):
                    for j in range(D_MODEL // 16):
                        sl = (b, t, i, pl.ds(j * 16, 16))
                        wbuf[sl] = rows_v[sl] * SCALE

            pltpu.async_copy(
                wbuf.at[b], out.at[pl.ds(mbase + c * G, G)], wsem
            )

            @pl.when(k < PAIRS - 1)
            def _refill():
                fire_gathers(c + 2, b, gsem)
        return carry

    lax.fori_loop(0, PAIRS, pair_body, 0)

    for b, wsem in ((0, ws0), (1, ws1)):
        pltpu.make_async_copy(
            wbuf.at[b], out.at[pl.ds(mbase, G)], wsem
        ).wait()


_emb = pl.kernel(
    _emb_body,
    out_type=jax.ShapeDtypeStruct((ROWS // 128, 128, D_MODEL), jnp.float32),
    mesh=plsc.VectorSubcoreMesh(core_axis_name="c", subcore_axis_name="s"),
    scratch_types=[
        pltpu.VMEM((MAJ_PER_W, 128), jnp.int32),
        pltpu.VMEM((2, G, 128, D_MODEL), jnp.float32),
        pltpu.VMEM((2, G, 128, D_MODEL), jnp.float32),
        pltpu.SemaphoreType.DMA,
        pltpu.SemaphoreType.DMA,
        pltpu.SemaphoreType.DMA,
        pltpu.SemaphoreType.DMA,
    ],
    compiler_params=pltpu.CompilerParams(use_tc_tiling_on_sc=False),
)


@jax.jit
def _run(x, table):
    idx = x.reshape(ROWS // 128, 128)
    out = _emb(table, idx)
    return out.reshape(x.shape[0], x.shape[1], D_MODEL)


def kernel(x, table):
    return _run(x, table)
